# baseline (speedup 1.0000x reference)
import functools

import jax
import jax.numpy as jnp
from jax.experimental import pallas as pl
from jax.experimental.pallas import tpu as pltpu

_ALPHA = 0.2
_BLK = 512
_EPS = 1e-5
_BF = jnp.bfloat16
_F32 = jnp.float32


def _elu(x):
    return jnp.where(x > 0, x, jnp.exp(x) - 1.0)


def _abcd(wh, an, h):
    f1 = jax.lax.dot_general(an[:h, :], wh, (((0,), (1,)), ((), ())),
                             preferred_element_type=_F32)
    f2 = jax.lax.dot_general(an[h:, :], wh, (((0,), (1,)), ((), ())),
                             preferred_element_type=_F32)
    mf2 = jnp.max(f2)
    t = f1 + mf2
    m = jnp.maximum(t, _ALPHA * t)
    a = jnp.exp(t - m)
    c = jnp.exp(_ALPHA * t - m)
    b = jnp.exp(f2 - mf2)
    d = jnp.exp(_ALPHA * (f2 - mf2))
    return a, c, b, d


def _side_prologue(x, wn, an, h, wa_s, wm_s, a_s, c_s, b_s, d_s):
    wh = jnp.dot(x, wn, preferred_element_type=_F32)
    wa_s[:, :h] = wh.astype(_BF)
    wa_s[:, h:] = jnp.ones_like(wa_s[:, h:])
    wm_s[...] = jnp.mean(wh, axis=0, keepdims=True)
    a, c, b, d = _abcd(wh, an, h)
    a_s[...] = jnp.transpose(a).astype(_BF)
    c_s[...] = jnp.transpose(c).astype(_BF)
    b_s[...] = b.astype(_BF)
    d_s[...] = d.astype(_BF)


def _att_blk(i, maskb, h, wa_s, wm_s, a_s, c_s, b_s, d_s):
    idx = pl.multiple_of(i * _BLK, _BLK)
    at = a_s[pl.ds(idx, _BLK), :]
    ct = c_s[pl.ds(idx, _BLK), :]
    p = maskb * jnp.maximum(at * b_s[...], ct * d_s[...])
    ha = jnp.dot(p, wa_s[...], preferred_element_type=_F32)
    den = ha[:, h:h + 1]
    return jnp.where(den > 0, ha[:, :h] / den, wm_s[...])


def _scratch_side(c, h):
    return [
        pltpu.VMEM((c, h + 1), _BF),
        pltpu.VMEM((1, h), _F32),
        pltpu.VMEM((c, 1), _BF),
        pltpu.VMEM((c, 1), _BF),
        pltpu.VMEM((1, c), _BF),
        pltpu.VMEM((1, c), _BF),
    ]


def _full(shape):
    return pl.BlockSpec(shape, lambda i: tuple(0 for _ in shape))


def _rowblk(cols, dtype=None):
    del dtype
    return pl.BlockSpec((_BLK, cols), lambda i: (i, 0))


def _l1_body(h, x_ref, wn_ref, an_ref, mask_ref, hp_ref, m8_ref,
             wa_s, wm_s, a_s, c_s, b_s, d_s):
    i = pl.program_id(0)

    @pl.when(i == 0)
    def _():
        _side_prologue(x_ref[...], wn_ref[...], an_ref[...], h,
                       wa_s, wm_s, a_s, c_s, b_s, d_s)

    hp_ref[...] = _att_blk(i, mask_ref[...].astype(_BF), h,
                           wa_s, wm_s, a_s, c_s, b_s, d_s)
    m8_ref[...] = mask_ref[...].astype(jnp.int8)


def _l1(mask, x, wn, an):
    r, cdim = mask.shape
    h = wn.shape[1]
    return pl.pallas_call(
        functools.partial(_l1_body, h),
        grid=(r // _BLK,),
        in_specs=[_full(x.shape), _full(wn.shape), _full(an.shape),
                  _rowblk(cdim)],
        out_specs=(_rowblk(h), _rowblk(cdim)),
        out_shape=(jax.ShapeDtypeStruct((r, h), _F32),
                   jax.ShapeDtypeStruct((r, cdim), jnp.int8)),
        scratch_shapes=_scratch_side(cdim, h),
    )(x, wn, an, mask)


def _l2n_body(h, h0, hp1_ref, ep1_ref, nea_ref, wn_ref, an_ref, mask_ref,
              hp2_ref, wa_s, wm_s, a_s, c_s, b_s, d_s, xo_s):
    i = pl.program_id(0)

    @pl.when(i == 0)
    def _():
        pooled = jnp.dot(nea_ref[...].astype(_BF),
                         ep1_ref[...].astype(_BF),
                         preferred_element_type=_F32)
        xo_s[:, :h0] = _elu(hp1_ref[...])
        xo_s[:, h0:] = _elu(pooled)
        _side_prologue(xo_s[...], wn_ref[...], an_ref[...], h,
                       wa_s, wm_s, a_s, c_s, b_s, d_s)

    hp2_ref[...] = _att_blk(i, mask_ref[...].astype(_BF), h,
                            wa_s, wm_s, a_s, c_s, b_s, d_s)


def _l2n(mask8, hp1, ep1, nea8, wn, an):
    r, cdim = mask8.shape
    h = wn.shape[1]
    h0 = hp1.shape[1]
    return pl.pallas_call(
        functools.partial(_l2n_body, h, h0),
        grid=(r // _BLK,),
        in_specs=[_full(hp1.shape), _full(ep1.shape), _full(nea8.shape),
                  _full(wn.shape), _full(an.shape), _rowblk(cdim)],
        out_specs=_rowblk(h),
        out_shape=jax.ShapeDtypeStruct((r, h), _F32),
        scratch_shapes=_scratch_side(cdim, h)
        + [pltpu.VMEM((r, 2 * h0), _F32)],
    )(hp1, ep1, nea8, wn, an, mask8)


def _l2e_body(h, h0, hp1_ref, ep1_ref, hp2_ref, nea_ref, we_ref, ae_ref,
              mask_ref, ep2_ref, eo2_ref,
              wa_s, wm_s, a_s, c_s, b_s, d_s, eo1_s, pl2_s):
    i = pl.program_id(0)

    @pl.when(i == 0)
    def _():
        neab = nea_ref[...].astype(_BF)
        pooled1 = jax.lax.dot_general(
            neab, hp1_ref[...].astype(_BF), (((0,), (0,)), ((), ())),
            preferred_element_type=_F32)
        eo1_s[:, :h0] = _elu(ep1_ref[...])
        eo1_s[:, h0:] = _elu(pooled1)
        _side_prologue(eo1_s[...], we_ref[...], ae_ref[...], h,
                       wa_s, wm_s, a_s, c_s, b_s, d_s)
        pl2_s[...] = jax.lax.dot_general(
            neab, hp2_ref[...].astype(_BF), (((0,), (0,)), ((), ())),
            preferred_element_type=_F32)

    ep2 = _att_blk(i, mask_ref[...].astype(_BF), h,
                   wa_s, wm_s, a_s, c_s, b_s, d_s)
    ep2_ref[...] = ep2
    idx = pl.multiple_of(i * _BLK, _BLK)
    eo2_ref[:, :h] = _elu(ep2)
    eo2_ref[:, h:] = _elu(pl2_s[pl.ds(idx, _BLK), :])


def _l2e(mask8, hp1, ep1, hp2, nea8, we, ae):
    r, cdim = mask8.shape
    h = we.shape[1]
    h0 = ep1.shape[1]
    return pl.pallas_call(
        functools.partial(_l2e_body, h, h0),
        grid=(r // _BLK,),
        in_specs=[_full(hp1.shape), _full(ep1.shape), _full(hp2.shape),
                  _full(nea8.shape), _full(we.shape), _full(ae.shape),
                  _rowblk(cdim)],
        out_specs=(_rowblk(h), _rowblk(2 * h)),
        out_shape=(jax.ShapeDtypeStruct((r, h), _F32),
                   jax.ShapeDtypeStruct((r, 2 * h), _F32)),
        scratch_shapes=_scratch_side(cdim, h)
        + [pltpu.VMEM((r, 2 * h0), _F32), pltpu.VMEM((r, h), _F32)],
    )(hp1, ep1, hp2, nea8, we, ae, mask8)


def _mlp_stats(pooled, w1_ref, b1_ref, w2_ref, b2_ref, z_ref, s_ref, ss_ref):
    t = jnp.maximum(
        jnp.dot(pooled, w1_ref[...], preferred_element_type=_F32)
        + b1_ref[...], 0.0)
    z = jnp.dot(t, w2_ref[...], preferred_element_type=_F32) + b2_ref[...]
    z_ref[...] = z

    @pl.when(pl.program_id(0) == 0)
    def _init():
        s_ref[...] = jnp.zeros_like(s_ref)
        ss_ref[...] = jnp.zeros_like(ss_ref)

    s_ref[...] += jnp.sum(z, axis=0, keepdims=True)
    ss_ref[...] += jnp.sum(z * z, axis=0, keepdims=True)


def _bn_relu_expr(z, s, ss, gamma, beta, n):
    mu = s / n
    var = ss / n - mu * mu
    return jnp.maximum((z - mu) / jnp.sqrt(var + _EPS) * gamma + beta, 0.0)


def _p1n_body(h2, a8_ref, hp2_ref, ep2_ref, nea_ref,
              w1_ref, b1_ref, w2_ref, b2_ref, z_ref, s_ref, ss_ref, h_s):
    i = pl.program_id(0)

    @pl.when(i == 0)
    def _():
        pooled = jnp.dot(nea_ref[...].astype(_BF),
                         ep2_ref[...].astype(_BF),
                         preferred_element_type=_F32)
        h_s[:, :h2] = _elu(hp2_ref[...]).astype(_BF)
        h_s[:, h2:] = _elu(pooled).astype(_BF)

    pooled = jnp.dot(a8_ref[...].astype(_BF), h_s[...],
                     preferred_element_type=_F32)
    _mlp_stats(pooled, w1_ref, b1_ref, w2_ref, b2_ref, z_ref, s_ref, ss_ref)


def _p1e_body(a8_ref, eo2_ref, w1_ref, b1_ref, w2_ref, b2_ref,
              z_ref, s_ref, ss_ref, h_s):
    i = pl.program_id(0)

    @pl.when(i == 0)
    def _():
        h_s[...] = eo2_ref[...].astype(_BF)

    pooled = jnp.dot(a8_ref[...].astype(_BF), h_s[...],
                     preferred_element_type=_F32)
    _mlp_stats(pooled, w1_ref, b1_ref, w2_ref, b2_ref, z_ref, s_ref, ss_ref)


def _p2_body(n, a8_ref, zp_ref, sp_ref, ssp_ref, g_ref, bt_ref,
             w1_ref, b1_ref, w2_ref, b2_ref, z_ref, s_ref, ss_ref, h_s):
    i = pl.program_id(0)

    @pl.when(i == 0)
    def _():
        h_s[...] = _bn_relu_expr(zp_ref[...], sp_ref[...], ssp_ref[...],
                                 g_ref[...], bt_ref[...], n).astype(_BF)

    pooled = jnp.dot(a8_ref[...].astype(_BF), h_s[...],
                     preferred_element_type=_F32)
    _mlp_stats(pooled, w1_ref, b1_ref, w2_ref, b2_ref, z_ref, s_ref, ss_ref)


def _pool_outs(r, k2):
    return (
        (
            pl.BlockSpec((_BLK, k2), lambda i: (i, 0)),
            pl.BlockSpec((1, k2), lambda i: (0, 0)),
            pl.BlockSpec((1, k2), lambda i: (0, 0)),
        ),
        (
            jax.ShapeDtypeStruct((r, k2), _F32),
            jax.ShapeDtypeStruct((1, k2), _F32),
            jax.ShapeDtypeStruct((1, k2), _F32),
        ),
    )


def _mlp_specs(mp):
    return [_full(mp['W1'].shape), _full((1, mp['W1'].shape[1])),
            _full(mp['W2'].shape), _full((1, mp['W2'].shape[1]))]


def _mlp_args(mp):
    return (mp['W1'], mp['b1'].reshape(1, -1), mp['W2'],
            mp['b2'].reshape(1, -1))


def _p1n(a8, hp2, ep2, nea8, mp):
    r, cdim = a8.shape
    h2 = hp2.shape[1]
    out_specs, out_shape = _pool_outs(r, mp['W2'].shape[1])
    return pl.pallas_call(
        functools.partial(_p1n_body, h2),
        grid=(r // _BLK,),
        in_specs=[_rowblk(cdim), _full(hp2.shape), _full(ep2.shape),
                  _full(nea8.shape)] + _mlp_specs(mp),
        out_specs=out_specs,
        out_shape=out_shape,
        scratch_shapes=[pltpu.VMEM((cdim, 2 * h2), _BF)],
    )(a8, hp2, ep2, nea8, *_mlp_args(mp))


def _p1e(a8, eo2, mp):
    r, cdim = a8.shape
    out_specs, out_shape = _pool_outs(r, mp['W2'].shape[1])
    return pl.pallas_call(
        _p1e_body,
        grid=(r // _BLK,),
        in_specs=[_rowblk(cdim), _full(eo2.shape)] + _mlp_specs(mp),
        out_specs=out_specs,
        out_shape=out_shape,
        scratch_shapes=[pltpu.VMEM(eo2.shape, _BF)],
    )(a8, eo2, *_mlp_args(mp))


def _p2(a8, zp, sp, ssp, bp, mp):
    r, cdim = a8.shape
    f = zp.shape[1]
    out_specs, out_shape = _pool_outs(r, mp['W2'].shape[1])
    return pl.pallas_call(
        functools.partial(_p2_body, float(cdim)),
        grid=(r // _BLK,),
        in_specs=[_rowblk(cdim), _full(zp.shape), _full((1, f)),
                  _full((1, f)), _full((1, f)), _full((1, f))]
        + _mlp_specs(mp),
        out_specs=out_specs,
        out_shape=out_shape,
        scratch_shapes=[pltpu.VMEM((cdim, f), _BF)],
    )(a8, zp, sp, ssp, bp['gamma'].reshape(1, -1),
      bp['beta'].reshape(1, -1), *_mlp_args(mp))


def _fin_body(h, n, z_ref, s_ref, ss_ref, g_ref, bt_ref, wn_ref, an_ref,
              mask_ref, o_ref, wa_s, wm_s, a_s, c_s, b_s, d_s, hbn_s):
    i = pl.program_id(0)

    @pl.when(i == 0)
    def _():
        hbn_s[...] = _bn_relu_expr(z_ref[...], s_ref[...], ss_ref[...],
                                   g_ref[...], bt_ref[...], n)
        _side_prologue(hbn_s[...], wn_ref[...], an_ref[...], h,
                       wa_s, wm_s, a_s, c_s, b_s, d_s)

    out = _att_blk(i, mask_ref[...].astype(_BF), h,
                   wa_s, wm_s, a_s, c_s, b_s, d_s)
    out = _elu(out)
    out = out - jnp.max(out, axis=1, keepdims=True)
    out = out - jnp.log(jnp.sum(jnp.exp(out), axis=1, keepdims=True))
    o_ref[...] = out


def _fin(mask8, z, s, ss, bp, wn, an):
    r, cdim = mask8.shape
    f = z.shape[1]
    h = wn.shape[1]
    return pl.pallas_call(
        functools.partial(_fin_body, h, float(r)),
        grid=(r // _BLK,),
        in_specs=[_full(z.shape), _full((1, f)), _full((1, f)),
                  _full((1, f)), _full((1, f)), _full(wn.shape),
                  _full(an.shape), _rowblk(cdim)],
        out_specs=_rowblk(h),
        out_shape=jax.ShapeDtypeStruct((r, h), _F32),
        scratch_shapes=_scratch_side(cdim, h)
        + [pltpu.VMEM((r, f), _F32)],
    )(z, s, ss, bp['gamma'].reshape(1, -1), bp['beta'].reshape(1, -1),
      wn, an, mask8)


def kernel(x, e_x, adj, e_adj, n_e_adj, params):
    nea8 = n_e_adj.astype(jnp.int8)
    p1, p2, p3 = params['in_att'], params['att0'], params['out_att']

    hp1, adj8 = _l1(adj, x, p1['Wn'], p1['an'])
    ep1, eadj8 = _l1(e_adj, e_x, p1['We'], p1['ae'])

    hp2 = _l2n(adj8, hp1, ep1, nea8, p2['Wn'], p2['an'])
    ep2, eo2 = _l2e(eadj8, hp1, ep1, hp2, nea8, p2['We'], p2['ae'])

    zn, sn, ssn = _p1n(adj8, hp2, ep2, nea8, params['mlp0'])
    ze, se, sse = _p1e(eadj8, eo2, params['mlp0'])
    zn, sn2, ssn2 = _p2(adj8, zn, sn, ssn, params['bn0'], params['mlp1'])
    ze, se2, sse2 = _p2(eadj8, ze, se, sse, params['bn0'], params['mlp1'])

    fx = _fin(adj8, zn, sn2, ssn2, params['bn1'], p3['Wn'], p3['an'])
    fe = _fin(eadj8, ze, se2, sse2, params['bn1'], p3['We'], p3['ae'])
    return fx, fe


# speedup vs baseline: 1.0643x; 1.0643x over previous
import functools

import jax
import jax.numpy as jnp
from jax.experimental import pallas as pl
from jax.experimental.pallas import tpu as pltpu

_ALPHA = 0.2
_BLK = 512
_EPS = 1e-5
_BF = jnp.bfloat16
_F32 = jnp.float32


def _elu(x):
    return jnp.where(x > 0, x, jnp.exp(x) - 1.0)


def _abcd(wh, an, h):
    f1 = jax.lax.dot_general(an[:h, :], wh, (((0,), (1,)), ((), ())),
                             preferred_element_type=_F32)
    f2 = jax.lax.dot_general(an[h:, :], wh, (((0,), (1,)), ((), ())),
                             preferred_element_type=_F32)
    mf2 = jnp.max(f2)
    t = f1 + mf2
    m = jnp.maximum(t, _ALPHA * t)
    a = jnp.exp(t - m)
    c = jnp.exp(_ALPHA * t - m)
    b = jnp.exp(f2 - mf2)
    d = jnp.exp(_ALPHA * (f2 - mf2))
    return a, c, b, d


def _side_prologue(x, wn, an, h, wa_s, wm_s, a_s, c_s, b_s, d_s):
    wh = jnp.dot(x, wn, preferred_element_type=_F32)
    wa_s[:, :h] = wh.astype(_BF)
    wa_s[:, h:] = jnp.ones_like(wa_s[:, h:])
    wm_s[...] = jnp.mean(wh, axis=0, keepdims=True)
    a, c, b, d = _abcd(wh, an, h)
    a_s[...] = jnp.transpose(a).astype(_BF)
    c_s[...] = jnp.transpose(c).astype(_BF)
    b_s[...] = b.astype(_BF)
    d_s[...] = d.astype(_BF)


def _att_blk(i, maskb, h, wa_s, wm_s, a_s, c_s, b_s, d_s):
    idx = pl.multiple_of(i * _BLK, _BLK)
    at = a_s[pl.ds(idx, _BLK), :]
    ct = c_s[pl.ds(idx, _BLK), :]
    p = maskb * jnp.maximum(at * b_s[...], ct * d_s[...])
    ha = jnp.dot(p, wa_s[...], preferred_element_type=_F32)
    den = ha[:, h:h + 1]
    return jnp.where(den > 0, ha[:, :h] / den, wm_s[...])


def _scratch_side(c, h):
    return [
        pltpu.VMEM((c, h + 1), _BF),
        pltpu.VMEM((1, h), _F32),
        pltpu.VMEM((c, 1), _BF),
        pltpu.VMEM((c, 1), _BF),
        pltpu.VMEM((1, c), _BF),
        pltpu.VMEM((1, c), _BF),
    ]


def _full(shape):
    return pl.BlockSpec(shape, lambda i: tuple(0 for _ in shape))


def _rowblk(cols):
    return pl.BlockSpec((_BLK, cols), lambda i: (i, 0))


def _attp_body(h, cast, x_ref, wn_ref, an_ref, mask_ref, hp_ref, *rest):
    if cast:
        m8_ref, scr = rest[0], rest[1:]
    else:
        scr = rest
    wa_s, wm_s, a_s, c_s, b_s, d_s = scr
    i = pl.program_id(0)

    @pl.when(i == 0)
    def _():
        _side_prologue(x_ref[...], wn_ref[...], an_ref[...], h,
                       wa_s, wm_s, a_s, c_s, b_s, d_s)

    hp_ref[...] = _att_blk(i, mask_ref[...].astype(_BF), h,
                           wa_s, wm_s, a_s, c_s, b_s, d_s)
    if cast:
        m8_ref[...] = mask_ref[...].astype(jnp.int8)


def _attp(mask, x, wn, an, cast=False):
    r, cdim = mask.shape
    h = wn.shape[1]
    out_specs = [_rowblk(h)]
    out_shape = [jax.ShapeDtypeStruct((r, h), _F32)]
    if cast:
        out_specs.append(_rowblk(cdim))
        out_shape.append(jax.ShapeDtypeStruct((r, cdim), jnp.int8))
    return pl.pallas_call(
        functools.partial(_attp_body, h, cast),
        grid=(r // _BLK,),
        in_specs=[_full(x.shape), _full(wn.shape), _full(an.shape),
                  _rowblk(cdim)],
        out_specs=tuple(out_specs),
        out_shape=tuple(out_shape),
        scratch_shapes=_scratch_side(cdim, h),
    )(x, wn, an, mask)


def _attcatp_body(h, cast, x_ref, wn_ref, an_ref, hp_ref, mask_ref,
                  nec_ref, ep_ref, eo_ref, *rest):
    if cast:
        m8_ref, scr = rest[0], rest[1:]
    else:
        scr = rest
    wa_s, wm_s, a_s, c_s, b_s, d_s = scr
    i = pl.program_id(0)

    @pl.when(i == 0)
    def _():
        _side_prologue(x_ref[...], wn_ref[...], an_ref[...], h,
                       wa_s, wm_s, a_s, c_s, b_s, d_s)

    ep = _att_blk(i, mask_ref[...].astype(_BF), h,
                  wa_s, wm_s, a_s, c_s, b_s, d_s)
    ep_ref[...] = ep
    pooled = jax.lax.dot_general(
        nec_ref[...].astype(_BF), hp_ref[...].astype(_BF),
        (((0,), (0,)), ((), ())), preferred_element_type=_F32)
    eo_ref[:, :h] = _elu(ep)
    eo_ref[:, h:] = _elu(pooled)
    if cast:
        m8_ref[...] = mask_ref[...].astype(jnp.int8)


def _attcatp(mask, x, wn, an, nea8, hp, cast=False):
    r, cdim = mask.shape
    n = nea8.shape[0]
    h = wn.shape[1]
    out_specs = [_rowblk(h), _rowblk(2 * h)]
    out_shape = [jax.ShapeDtypeStruct((r, h), _F32),
                 jax.ShapeDtypeStruct((r, 2 * h), _F32)]
    if cast:
        out_specs.append(_rowblk(cdim))
        out_shape.append(jax.ShapeDtypeStruct((r, cdim), jnp.int8))
    return pl.pallas_call(
        functools.partial(_attcatp_body, h, cast),
        grid=(r // _BLK,),
        in_specs=[_full(x.shape), _full(wn.shape), _full(an.shape),
                  _full(hp.shape), _rowblk(cdim),
                  pl.BlockSpec((n, _BLK), lambda i: (0, i))],
        out_specs=tuple(out_specs),
        out_shape=tuple(out_shape),
        scratch_shapes=_scratch_side(cdim, h),
    )(x, wn, an, hp, mask, nea8)


def _cat_body(h, ne_ref, hp_ref, ep_ref, o_ref):
    pooled = jnp.dot(ne_ref[...].astype(_BF), ep_ref[...].astype(_BF),
                     preferred_element_type=_F32)
    o_ref[:, :h] = _elu(hp_ref[...])
    o_ref[:, h:] = _elu(pooled)


def _cat(ne8, hp, ep):
    r, cdim = ne8.shape
    h = hp.shape[1]
    return pl.pallas_call(
        functools.partial(_cat_body, h),
        grid=(r // _BLK,),
        in_specs=[_rowblk(cdim), _rowblk(h), _full(ep.shape)],
        out_specs=_rowblk(2 * h),
        out_shape=jax.ShapeDtypeStruct((r, 2 * h), _F32),
    )(ne8, hp, ep)


def _mlp_stats(pooled, w1_ref, b1_ref, w2_ref, b2_ref, z_ref, s_ref, ss_ref):
    t = jnp.maximum(
        jnp.dot(pooled, w1_ref[...], preferred_element_type=_F32)
        + b1_ref[...], 0.0)
    z = jnp.dot(t, w2_ref[...], preferred_element_type=_F32) + b2_ref[...]
    z_ref[...] = z

    @pl.when(pl.program_id(0) == 0)
    def _init():
        s_ref[...] = jnp.zeros_like(s_ref)
        ss_ref[...] = jnp.zeros_like(ss_ref)

    s_ref[...] += jnp.sum(z, axis=0, keepdims=True)
    ss_ref[...] += jnp.sum(z * z, axis=0, keepdims=True)


def _bn_relu_expr(z, s, ss, gamma, beta, n):
    mu = s / n
    var = ss / n - mu * mu
    return jnp.maximum((z - mu) / jnp.sqrt(var + _EPS) * gamma + beta, 0.0)


def _p1_body(a8_ref, h_ref, w1_ref, b1_ref, w2_ref, b2_ref,
             z_ref, s_ref, ss_ref, h_s):
    i = pl.program_id(0)

    @pl.when(i == 0)
    def _():
        h_s[...] = h_ref[...].astype(_BF)

    pooled = jnp.dot(a8_ref[...].astype(_BF), h_s[...],
                     preferred_element_type=_F32)
    _mlp_stats(pooled, w1_ref, b1_ref, w2_ref, b2_ref, z_ref, s_ref, ss_ref)


def _p2_body(n, a8_ref, zp_ref, sp_ref, ssp_ref, g_ref, bt_ref,
             w1_ref, b1_ref, w2_ref, b2_ref, z_ref, s_ref, ss_ref, h_s):
    i = pl.program_id(0)

    @pl.when(i == 0)
    def _():
        h_s[...] = _bn_relu_expr(zp_ref[...], sp_ref[...], ssp_ref[...],
                                 g_ref[...], bt_ref[...], n).astype(_BF)

    pooled = jnp.dot(a8_ref[...].astype(_BF), h_s[...],
                     preferred_element_type=_F32)
    _mlp_stats(pooled, w1_ref, b1_ref, w2_ref, b2_ref, z_ref, s_ref, ss_ref)


def _pool_outs(r, k2):
    return (
        (
            pl.BlockSpec((_BLK, k2), lambda i: (i, 0)),
            pl.BlockSpec((1, k2), lambda i: (0, 0)),
            pl.BlockSpec((1, k2), lambda i: (0, 0)),
        ),
        (
            jax.ShapeDtypeStruct((r, k2), _F32),
            jax.ShapeDtypeStruct((1, k2), _F32),
            jax.ShapeDtypeStruct((1, k2), _F32),
        ),
    )


def _mlp_specs(mp):
    return [_full(mp['W1'].shape), _full((1, mp['W1'].shape[1])),
            _full(mp['W2'].shape), _full((1, mp['W2'].shape[1]))]


def _mlp_args(mp):
    return (mp['W1'], mp['b1'].reshape(1, -1), mp['W2'],
            mp['b2'].reshape(1, -1))


def _p1(a8, h, mp):
    r, cdim = a8.shape
    out_specs, out_shape = _pool_outs(r, mp['W2'].shape[1])
    return pl.pallas_call(
        _p1_body,
        grid=(r // _BLK,),
        in_specs=[_rowblk(cdim), _full(h.shape)] + _mlp_specs(mp),
        out_specs=out_specs,
        out_shape=out_shape,
        scratch_shapes=[pltpu.VMEM(h.shape, _BF)],
    )(a8, h, *_mlp_args(mp))


def _p2(a8, zp, sp, ssp, bp, mp):
    r, cdim = a8.shape
    f = zp.shape[1]
    out_specs, out_shape = _pool_outs(r, mp['W2'].shape[1])
    return pl.pallas_call(
        functools.partial(_p2_body, float(cdim)),
        grid=(r // _BLK,),
        in_specs=[_rowblk(cdim), _full(zp.shape), _full((1, f)),
                  _full((1, f)), _full((1, f)), _full((1, f))]
        + _mlp_specs(mp),
        out_specs=out_specs,
        out_shape=out_shape,
        scratch_shapes=[pltpu.VMEM((cdim, f), _BF)],
    )(a8, zp, sp, ssp, bp['gamma'].reshape(1, -1),
      bp['beta'].reshape(1, -1), *_mlp_args(mp))


def _fin_body(h, n, z_ref, s_ref, ss_ref, g_ref, bt_ref, wn_ref, an_ref,
              mask_ref, o_ref, wa_s, wm_s, a_s, c_s, b_s, d_s, hbn_s):
    i = pl.program_id(0)

    @pl.when(i == 0)
    def _():
        hbn_s[...] = _bn_relu_expr(z_ref[...], s_ref[...], ss_ref[...],
                                   g_ref[...], bt_ref[...], n)
        _side_prologue(hbn_s[...], wn_ref[...], an_ref[...], h,
                       wa_s, wm_s, a_s, c_s, b_s, d_s)

    out = _att_blk(i, mask_ref[...].astype(_BF), h,
                   wa_s, wm_s, a_s, c_s, b_s, d_s)
    out = _elu(out)
    out = out - jnp.max(out, axis=1, keepdims=True)
    out = out - jnp.log(jnp.sum(jnp.exp(out), axis=1, keepdims=True))
    o_ref[...] = out


def _fin(mask8, z, s, ss, bp, wn, an):
    r, cdim = mask8.shape
    f = z.shape[1]
    h = wn.shape[1]
    return pl.pallas_call(
        functools.partial(_fin_body, h, float(r)),
        grid=(r // _BLK,),
        in_specs=[_full(z.shape), _full((1, f)), _full((1, f)),
                  _full((1, f)), _full((1, f)), _full(wn.shape),
                  _full(an.shape), _rowblk(cdim)],
        out_specs=_rowblk(h),
        out_shape=jax.ShapeDtypeStruct((r, h), _F32),
        scratch_shapes=_scratch_side(cdim, h)
        + [pltpu.VMEM((r, f), _F32)],
    )(z, s, ss, bp['gamma'].reshape(1, -1), bp['beta'].reshape(1, -1),
      wn, an, mask8)


def kernel(x, e_x, adj, e_adj, n_e_adj, params):
    nea8 = n_e_adj.astype(jnp.int8)
    p1, p2, p3 = params['in_att'], params['att0'], params['out_att']

    hp1, adj8 = _attp(adj, x, p1['Wn'], p1['an'], cast=True)
    ep1, eo1, eadj8 = _attcatp(e_adj, e_x, p1['We'], p1['ae'], nea8, hp1,
                               cast=True)
    xo1 = _cat(nea8, hp1, ep1)

    hp2, = _attp(adj8, xo1, p2['Wn'], p2['an'])
    ep2, eo2 = _attcatp(eadj8, eo1, p2['We'], p2['ae'], nea8, hp2)
    xo2 = _cat(nea8, hp2, ep2)

    zn, sn, ssn = _p1(adj8, xo2, params['mlp0'])
    ze, se, sse = _p1(eadj8, eo2, params['mlp0'])
    zn, sn2, ssn2 = _p2(adj8, zn, sn, ssn, params['bn0'], params['mlp1'])
    ze, se2, sse2 = _p2(eadj8, ze, se, sse, params['bn0'], params['mlp1'])

    fx = _fin(adj8, zn, sn2, ssn2, params['bn1'], p3['Wn'], p3['an'])
    fe = _fin(eadj8, ze, se2, sse2, params['bn1'], p3['We'], p3['ae'])
    return fx, fe


# speedup vs baseline: 1.1205x; 1.0528x over previous
import functools

import jax
import jax.numpy as jnp
from jax.experimental import pallas as pl
from jax.experimental.pallas import tpu as pltpu

_ALPHA = 0.2
_BLK = 512
_EPS = 1e-5
_BF = jnp.bfloat16
_F32 = jnp.float32


def _elu(x):
    return jnp.where(x > 0, x, jnp.exp(x) - 1.0)


def _abcd(wh, an, h):
    f1 = jax.lax.dot_general(an[:h, :], wh, (((0,), (1,)), ((), ())),
                             preferred_element_type=_F32)
    f2 = jax.lax.dot_general(an[h:, :], wh, (((0,), (1,)), ((), ())),
                             preferred_element_type=_F32)
    mf2 = jnp.max(f2)
    t = f1 + mf2
    m = jnp.maximum(t, _ALPHA * t)
    a = jnp.exp(t - m)
    c = jnp.exp(_ALPHA * t - m)
    b = jnp.exp(f2 - mf2)
    d = jnp.exp(_ALPHA * (f2 - mf2))
    return a, c, b, d


def _side_prologue(x, wn, an, h, wa_s, wm_s, a_s, c_s, b_s, d_s):
    wh = jnp.dot(x, wn, preferred_element_type=_F32)
    wa_s[:, :h] = wh.astype(_BF)
    wa_s[:, h:] = jnp.ones_like(wa_s[:, h:])
    wm_s[...] = jnp.mean(wh, axis=0, keepdims=True)
    a, c, b, d = _abcd(wh, an, h)
    a_s[...] = jnp.transpose(a).astype(_BF)
    c_s[...] = jnp.transpose(c).astype(_BF)
    b_s[...] = b.astype(_BF)
    d_s[...] = d.astype(_BF)


def _att_blk(i, maskb, h, wa_s, wm_s, a_s, c_s, b_s, d_s):
    idx = pl.multiple_of(i * _BLK, _BLK)
    at = a_s[pl.ds(idx, _BLK), :]
    ct = c_s[pl.ds(idx, _BLK), :]
    p = maskb * jnp.maximum(at * b_s[...], ct * d_s[...])
    ha = jnp.dot(p, wa_s[...], preferred_element_type=_F32)
    den = ha[:, h:h + 1]
    return jnp.where(den > 0, ha[:, :h] / den, wm_s[...])


def _scratch_side(c, h):
    return [
        pltpu.VMEM((c, h + 1), _BF),
        pltpu.VMEM((1, h), _F32),
        pltpu.VMEM((c, 1), _BF),
        pltpu.VMEM((c, 1), _BF),
        pltpu.VMEM((1, c), _BF),
        pltpu.VMEM((1, c), _BF),
    ]


def _full(shape):
    return pl.BlockSpec(shape, lambda i: tuple(0 for _ in shape))


def _rowblk(cols):
    return pl.BlockSpec((_BLK, cols), lambda i: (i, 0))


def _attp_body(h, cast, x_ref, wn_ref, an_ref, mask_ref, hp_ref, *rest):
    if cast:
        m8_ref, scr = rest[0], rest[1:]
    else:
        scr = rest
    wa_s, wm_s, a_s, c_s, b_s, d_s = scr
    i = pl.program_id(0)

    @pl.when(i == 0)
    def _():
        _side_prologue(x_ref[...], wn_ref[...], an_ref[...], h,
                       wa_s, wm_s, a_s, c_s, b_s, d_s)

    hp_ref[...] = _att_blk(i, mask_ref[...].astype(_BF), h,
                           wa_s, wm_s, a_s, c_s, b_s, d_s)
    if cast:
        m8_ref[...] = mask_ref[...].astype(jnp.int4)


def _attp(mask, x, wn, an, cast=False):
    r, cdim = mask.shape
    h = wn.shape[1]
    out_specs = [_rowblk(h)]
    out_shape = [jax.ShapeDtypeStruct((r, h), _F32)]
    if cast:
        out_specs.append(_rowblk(cdim))
        out_shape.append(jax.ShapeDtypeStruct((r, cdim), jnp.int4))
    return pl.pallas_call(
        functools.partial(_attp_body, h, cast),
        grid=(r // _BLK,),
        in_specs=[_full(x.shape), _full(wn.shape), _full(an.shape),
                  _rowblk(cdim)],
        out_specs=tuple(out_specs),
        out_shape=tuple(out_shape),
        scratch_shapes=_scratch_side(cdim, h),
    )(x, wn, an, mask)


def _attcatp_body(h, cast, x_ref, wn_ref, an_ref, hp_ref, mask_ref,
                  nec_ref, ep_ref, eo_ref, *rest):
    if cast:
        m8_ref, scr = rest[0], rest[1:]
    else:
        scr = rest
    wa_s, wm_s, a_s, c_s, b_s, d_s = scr
    i = pl.program_id(0)

    @pl.when(i == 0)
    def _():
        _side_prologue(x_ref[...], wn_ref[...], an_ref[...], h,
                       wa_s, wm_s, a_s, c_s, b_s, d_s)

    ep = _att_blk(i, mask_ref[...].astype(_BF), h,
                  wa_s, wm_s, a_s, c_s, b_s, d_s)
    ep_ref[...] = ep
    pooled = jax.lax.dot_general(
        nec_ref[...].astype(_BF), hp_ref[...].astype(_BF),
        (((0,), (0,)), ((), ())), preferred_element_type=_F32)
    eo_ref[:, :h] = _elu(ep)
    eo_ref[:, h:] = _elu(pooled)
    if cast:
        m8_ref[...] = mask_ref[...].astype(jnp.int4)


def _attcatp(mask, x, wn, an, nea8, hp, cast=False):
    r, cdim = mask.shape
    n = nea8.shape[0]
    h = wn.shape[1]
    out_specs = [_rowblk(h), _rowblk(2 * h)]
    out_shape = [jax.ShapeDtypeStruct((r, h), _F32),
                 jax.ShapeDtypeStruct((r, 2 * h), _F32)]
    if cast:
        out_specs.append(_rowblk(cdim))
        out_shape.append(jax.ShapeDtypeStruct((r, cdim), jnp.int4))
    return pl.pallas_call(
        functools.partial(_attcatp_body, h, cast),
        grid=(r // _BLK,),
        in_specs=[_full(x.shape), _full(wn.shape), _full(an.shape),
                  _full(hp.shape), _rowblk(cdim),
                  pl.BlockSpec((n, _BLK), lambda i: (0, i))],
        out_specs=tuple(out_specs),
        out_shape=tuple(out_shape),
        scratch_shapes=_scratch_side(cdim, h),
    )(x, wn, an, hp, mask, nea8)


def _cat_body(h, ne_ref, hp_ref, ep_ref, o_ref):
    pooled = jnp.dot(ne_ref[...].astype(_BF), ep_ref[...].astype(_BF),
                     preferred_element_type=_F32)
    o_ref[:, :h] = _elu(hp_ref[...])
    o_ref[:, h:] = _elu(pooled)


def _cat(ne8, hp, ep):
    r, cdim = ne8.shape
    h = hp.shape[1]
    return pl.pallas_call(
        functools.partial(_cat_body, h),
        grid=(r // _BLK,),
        in_specs=[_rowblk(cdim), _rowblk(h), _full(ep.shape)],
        out_specs=_rowblk(2 * h),
        out_shape=jax.ShapeDtypeStruct((r, 2 * h), _F32),
    )(ne8, hp, ep)


def _mlp_stats(pooled, w1_ref, b1_ref, w2_ref, b2_ref, z_ref, s_ref, ss_ref):
    t = jnp.maximum(
        jnp.dot(pooled, w1_ref[...], preferred_element_type=_F32)
        + b1_ref[...], 0.0)
    z = jnp.dot(t, w2_ref[...], preferred_element_type=_F32) + b2_ref[...]
    z_ref[...] = z

    @pl.when(pl.program_id(0) == 0)
    def _init():
        s_ref[...] = jnp.zeros_like(s_ref)
        ss_ref[...] = jnp.zeros_like(ss_ref)

    s_ref[...] += jnp.sum(z, axis=0, keepdims=True)
    ss_ref[...] += jnp.sum(z * z, axis=0, keepdims=True)


def _bn_relu_expr(z, s, ss, gamma, beta, n):
    mu = s / n
    var = ss / n - mu * mu
    return jnp.maximum((z - mu) / jnp.sqrt(var + _EPS) * gamma + beta, 0.0)


def _p1_body(a8_ref, h_ref, w1_ref, b1_ref, w2_ref, b2_ref,
             z_ref, s_ref, ss_ref, h_s):
    i = pl.program_id(0)

    @pl.when(i == 0)
    def _():
        h_s[...] = h_ref[...].astype(_BF)

    pooled = jnp.dot(a8_ref[...].astype(_BF), h_s[...],
                     preferred_element_type=_F32)
    _mlp_stats(pooled, w1_ref, b1_ref, w2_ref, b2_ref, z_ref, s_ref, ss_ref)


def _p2_body(n, a8_ref, zp_ref, sp_ref, ssp_ref, g_ref, bt_ref,
             w1_ref, b1_ref, w2_ref, b2_ref, z_ref, s_ref, ss_ref, h_s):
    i = pl.program_id(0)

    @pl.when(i == 0)
    def _():
        h_s[...] = _bn_relu_expr(zp_ref[...], sp_ref[...], ssp_ref[...],
                                 g_ref[...], bt_ref[...], n).astype(_BF)

    pooled = jnp.dot(a8_ref[...].astype(_BF), h_s[...],
                     preferred_element_type=_F32)
    _mlp_stats(pooled, w1_ref, b1_ref, w2_ref, b2_ref, z_ref, s_ref, ss_ref)


def _pool_outs(r, k2):
    return (
        (
            pl.BlockSpec((_BLK, k2), lambda i: (i, 0)),
            pl.BlockSpec((1, k2), lambda i: (0, 0)),
            pl.BlockSpec((1, k2), lambda i: (0, 0)),
        ),
        (
            jax.ShapeDtypeStruct((r, k2), _F32),
            jax.ShapeDtypeStruct((1, k2), _F32),
            jax.ShapeDtypeStruct((1, k2), _F32),
        ),
    )


def _mlp_specs(mp):
    return [_full(mp['W1'].shape), _full((1, mp['W1'].shape[1])),
            _full(mp['W2'].shape), _full((1, mp['W2'].shape[1]))]


def _mlp_args(mp):
    return (mp['W1'], mp['b1'].reshape(1, -1), mp['W2'],
            mp['b2'].reshape(1, -1))


def _p1(a8, h, mp):
    r, cdim = a8.shape
    out_specs, out_shape = _pool_outs(r, mp['W2'].shape[1])
    return pl.pallas_call(
        _p1_body,
        grid=(r // _BLK,),
        in_specs=[_rowblk(cdim), _full(h.shape)] + _mlp_specs(mp),
        out_specs=out_specs,
        out_shape=out_shape,
        scratch_shapes=[pltpu.VMEM(h.shape, _BF)],
    )(a8, h, *_mlp_args(mp))


def _p2(a8, zp, sp, ssp, bp, mp):
    r, cdim = a8.shape
    f = zp.shape[1]
    out_specs, out_shape = _pool_outs(r, mp['W2'].shape[1])
    return pl.pallas_call(
        functools.partial(_p2_body, float(cdim)),
        grid=(r // _BLK,),
        in_specs=[_rowblk(cdim), _full(zp.shape), _full((1, f)),
                  _full((1, f)), _full((1, f)), _full((1, f))]
        + _mlp_specs(mp),
        out_specs=out_specs,
        out_shape=out_shape,
        scratch_shapes=[pltpu.VMEM((cdim, f), _BF)],
    )(a8, zp, sp, ssp, bp['gamma'].reshape(1, -1),
      bp['beta'].reshape(1, -1), *_mlp_args(mp))


def _fin_body(h, n, z_ref, s_ref, ss_ref, g_ref, bt_ref, wn_ref, an_ref,
              mask_ref, o_ref, wa_s, wm_s, a_s, c_s, b_s, d_s, hbn_s):
    i = pl.program_id(0)

    @pl.when(i == 0)
    def _():
        hbn_s[...] = _bn_relu_expr(z_ref[...], s_ref[...], ss_ref[...],
                                   g_ref[...], bt_ref[...], n)
        _side_prologue(hbn_s[...], wn_ref[...], an_ref[...], h,
                       wa_s, wm_s, a_s, c_s, b_s, d_s)

    out = _att_blk(i, mask_ref[...].astype(_BF), h,
                   wa_s, wm_s, a_s, c_s, b_s, d_s)
    out = _elu(out)
    out = out - jnp.max(out, axis=1, keepdims=True)
    out = out - jnp.log(jnp.sum(jnp.exp(out), axis=1, keepdims=True))
    o_ref[...] = out


def _fin(mask8, z, s, ss, bp, wn, an):
    r, cdim = mask8.shape
    f = z.shape[1]
    h = wn.shape[1]
    return pl.pallas_call(
        functools.partial(_fin_body, h, float(r)),
        grid=(r // _BLK,),
        in_specs=[_full(z.shape), _full((1, f)), _full((1, f)),
                  _full((1, f)), _full((1, f)), _full(wn.shape),
                  _full(an.shape), _rowblk(cdim)],
        out_specs=_rowblk(h),
        out_shape=jax.ShapeDtypeStruct((r, h), _F32),
        scratch_shapes=_scratch_side(cdim, h)
        + [pltpu.VMEM((r, f), _F32)],
    )(z, s, ss, bp['gamma'].reshape(1, -1), bp['beta'].reshape(1, -1),
      wn, an, mask8)


def kernel(x, e_x, adj, e_adj, n_e_adj, params):
    nea8 = n_e_adj.astype(jnp.int4)
    p1, p2, p3 = params['in_att'], params['att0'], params['out_att']

    hp1, adj8 = _attp(adj, x, p1['Wn'], p1['an'], cast=True)
    ep1, eo1, eadj8 = _attcatp(e_adj, e_x, p1['We'], p1['ae'], nea8, hp1,
                               cast=True)
    xo1 = _cat(nea8, hp1, ep1)

    hp2, = _attp(adj8, xo1, p2['Wn'], p2['an'])
    ep2, eo2 = _attcatp(eadj8, eo1, p2['We'], p2['ae'], nea8, hp2)
    xo2 = _cat(nea8, hp2, ep2)

    zn, sn, ssn = _p1(adj8, xo2, params['mlp0'])
    ze, se, sse = _p1(eadj8, eo2, params['mlp0'])
    zn, sn2, ssn2 = _p2(adj8, zn, sn, ssn, params['bn0'], params['mlp1'])
    ze, se2, sse2 = _p2(eadj8, ze, se, sse, params['bn0'], params['mlp1'])

    fx = _fin(adj8, zn, sn2, ssn2, params['bn1'], p3['Wn'], p3['an'])
    fe = _fin(eadj8, ze, se2, sse2, params['bn1'], p3['We'], p3['ae'])
    return fx, fe


# speedup vs baseline: 1.1347x; 1.0127x over previous
import functools

import jax
import jax.numpy as jnp
from jax.experimental import pallas as pl
from jax.experimental.pallas import tpu as pltpu

_ALPHA = 0.2
_BLK = 512
_EPS = 1e-5
_BF = jnp.bfloat16
_F32 = jnp.float32


def _elu(x):
    return jnp.where(x > 0, x, jnp.exp(x) - 1.0)


def _abcd(wh, an, h):
    f1 = jax.lax.dot_general(an[:h, :], wh, (((0,), (1,)), ((), ())),
                             preferred_element_type=_F32)
    f2 = jax.lax.dot_general(an[h:, :], wh, (((0,), (1,)), ((), ())),
                             preferred_element_type=_F32)
    mf2 = jnp.max(f2)
    t = f1 + mf2
    m = jnp.maximum(t, _ALPHA * t)
    a = jnp.exp(t - m)
    c = jnp.exp(_ALPHA * t - m)
    b = jnp.exp(f2 - mf2)
    d = jnp.exp(_ALPHA * (f2 - mf2))
    return a, c, b, d


def _side_prologue(x, wn, an, h, wa_s, wm_s, a_s, c_s, b_s, d_s):
    wh = jnp.dot(x, wn, preferred_element_type=_F32)
    wa_s[:, :h] = wh.astype(_BF)
    wa_s[:, h:] = jnp.ones_like(wa_s[:, h:])
    wm_s[...] = jnp.mean(wh, axis=0, keepdims=True)
    a, c, b, d = _abcd(wh, an, h)
    a_s[...] = jnp.transpose(a).astype(_BF)
    c_s[...] = jnp.transpose(c).astype(_BF)
    b_s[...] = b.astype(_BF)
    d_s[...] = d.astype(_BF)


def _att_blk(i, maskb, h, blk, wa_s, wm_s, a_s, c_s, b_s, d_s):
    idx = pl.multiple_of(i * blk, blk)
    at = a_s[pl.ds(idx, blk), :]
    ct = c_s[pl.ds(idx, blk), :]
    p = maskb * jnp.maximum(at * b_s[...], ct * d_s[...])
    ha = jnp.dot(p, wa_s[...], preferred_element_type=_F32)
    den = ha[:, h:h + 1]
    return jnp.where(den > 0, ha[:, :h] / den, wm_s[...])


def _scratch_side(c, h):
    return [
        pltpu.VMEM((c, h + 1), _BF),
        pltpu.VMEM((1, h), _F32),
        pltpu.VMEM((c, 1), _BF),
        pltpu.VMEM((c, 1), _BF),
        pltpu.VMEM((1, c), _BF),
        pltpu.VMEM((1, c), _BF),
    ]


def _full(shape):
    return pl.BlockSpec(shape, lambda i: tuple(0 for _ in shape))


def _rowblk(cols, blk=_BLK):
    return pl.BlockSpec((blk, cols), lambda i: (i, 0))


def _attp_body(h, cast, blk, x_ref, wn_ref, an_ref, mask_ref, hp_ref, *rest):
    if cast:
        m8_ref, scr = rest[0], rest[1:]
    else:
        scr = rest
    wa_s, wm_s, a_s, c_s, b_s, d_s = scr
    i = pl.program_id(0)

    @pl.when(i == 0)
    def _():
        _side_prologue(x_ref[...], wn_ref[...], an_ref[...], h,
                       wa_s, wm_s, a_s, c_s, b_s, d_s)

    hp_ref[...] = _att_blk(i, mask_ref[...].astype(_BF), h, blk,
                           wa_s, wm_s, a_s, c_s, b_s, d_s)
    if cast:
        m8_ref[...] = mask_ref[...].astype(jnp.int4)


def _attp(mask, x, wn, an, cast=False, blk=_BLK):
    r, cdim = mask.shape
    h = wn.shape[1]
    out_specs = [_rowblk(h, blk)]
    out_shape = [jax.ShapeDtypeStruct((r, h), _F32)]
    if cast:
        out_specs.append(_rowblk(cdim, blk))
        out_shape.append(jax.ShapeDtypeStruct((r, cdim), jnp.int4))
    return pl.pallas_call(
        functools.partial(_attp_body, h, cast, blk),
        grid=(r // blk,),
        in_specs=[_full(x.shape), _full(wn.shape), _full(an.shape),
                  _rowblk(cdim, blk)],
        out_specs=tuple(out_specs),
        out_shape=tuple(out_shape),
        scratch_shapes=_scratch_side(cdim, h),
    )(x, wn, an, mask)


def _attcatp_body(h, cast, blk, x_ref, wn_ref, an_ref, hp_ref, mask_ref,
                  nec_ref, ep_ref, eo_ref, *rest):
    if cast:
        m8_ref, scr = rest[0], rest[1:]
    else:
        scr = rest
    wa_s, wm_s, a_s, c_s, b_s, d_s = scr
    i = pl.program_id(0)

    @pl.when(i == 0)
    def _():
        _side_prologue(x_ref[...], wn_ref[...], an_ref[...], h,
                       wa_s, wm_s, a_s, c_s, b_s, d_s)

    ep = _att_blk(i, mask_ref[...].astype(_BF), h, blk,
                  wa_s, wm_s, a_s, c_s, b_s, d_s)
    ep_ref[...] = ep
    pooled = jax.lax.dot_general(
        nec_ref[...].astype(_BF), hp_ref[...].astype(_BF),
        (((0,), (0,)), ((), ())), preferred_element_type=_F32)
    eo_ref[:, :h] = _elu(ep)
    eo_ref[:, h:] = _elu(pooled)
    if cast:
        m8_ref[...] = mask_ref[...].astype(jnp.int4)


def _attcatp(mask, x, wn, an, nea8, hp, cast=False, blk=_BLK):
    r, cdim = mask.shape
    n = nea8.shape[0]
    h = wn.shape[1]
    out_specs = [_rowblk(h, blk), _rowblk(2 * h, blk)]
    out_shape = [jax.ShapeDtypeStruct((r, h), _F32),
                 jax.ShapeDtypeStruct((r, 2 * h), _F32)]
    if cast:
        out_specs.append(_rowblk(cdim, blk))
        out_shape.append(jax.ShapeDtypeStruct((r, cdim), jnp.int4))
    return pl.pallas_call(
        functools.partial(_attcatp_body, h, cast, blk),
        grid=(r // blk,),
        in_specs=[_full(x.shape), _full(wn.shape), _full(an.shape),
                  _full(hp.shape), _rowblk(cdim, blk),
                  pl.BlockSpec((n, blk), lambda i: (0, i))],
        out_specs=tuple(out_specs),
        out_shape=tuple(out_shape),
        scratch_shapes=_scratch_side(cdim, h),
    )(x, wn, an, hp, mask, nea8)


def _cat_body(h, ne_ref, hp_ref, ep_ref, o_ref):
    pooled = jnp.dot(ne_ref[...].astype(_BF), ep_ref[...].astype(_BF),
                     preferred_element_type=_F32)
    o_ref[:, :h] = _elu(hp_ref[...])
    o_ref[:, h:] = _elu(pooled)


def _cat(ne8, hp, ep, blk=_BLK):
    r, cdim = ne8.shape
    h = hp.shape[1]
    return pl.pallas_call(
        functools.partial(_cat_body, h),
        grid=(r // blk,),
        in_specs=[_rowblk(cdim, blk), _rowblk(h, blk), _full(ep.shape)],
        out_specs=_rowblk(2 * h, blk),
        out_shape=jax.ShapeDtypeStruct((r, 2 * h), _F32),
    )(ne8, hp, ep)


def _mlp_stats(pooled, w1_ref, b1_ref, w2_ref, b2_ref, z_ref, s_ref, ss_ref):
    t = jnp.maximum(
        jnp.dot(pooled, w1_ref[...], preferred_element_type=_F32)
        + b1_ref[...], 0.0)
    z = jnp.dot(t, w2_ref[...], preferred_element_type=_F32) + b2_ref[...]
    z_ref[...] = z

    @pl.when(pl.program_id(0) == 0)
    def _init():
        s_ref[...] = jnp.zeros_like(s_ref)
        ss_ref[...] = jnp.zeros_like(ss_ref)

    s_ref[...] += jnp.sum(z, axis=0, keepdims=True)
    ss_ref[...] += jnp.sum(z * z, axis=0, keepdims=True)


def _bn_relu_expr(z, s, ss, gamma, beta, n):
    mu = s / n
    var = ss / n - mu * mu
    return jnp.maximum((z - mu) / jnp.sqrt(var + _EPS) * gamma + beta, 0.0)


def _p1_body(a8_ref, h_ref, w1_ref, b1_ref, w2_ref, b2_ref,
             z_ref, s_ref, ss_ref, h_s):
    i = pl.program_id(0)

    @pl.when(i == 0)
    def _():
        h_s[...] = h_ref[...].astype(_BF)

    pooled = jnp.dot(a8_ref[...].astype(_BF), h_s[...],
                     preferred_element_type=_F32)
    _mlp_stats(pooled, w1_ref, b1_ref, w2_ref, b2_ref, z_ref, s_ref, ss_ref)


def _p2_body(n, a8_ref, zp_ref, sp_ref, ssp_ref, g_ref, bt_ref,
             w1_ref, b1_ref, w2_ref, b2_ref, z_ref, s_ref, ss_ref, h_s):
    i = pl.program_id(0)

    @pl.when(i == 0)
    def _():
        h_s[...] = _bn_relu_expr(zp_ref[...], sp_ref[...], ssp_ref[...],
                                 g_ref[...], bt_ref[...], n).astype(_BF)

    pooled = jnp.dot(a8_ref[...].astype(_BF), h_s[...],
                     preferred_element_type=_F32)
    _mlp_stats(pooled, w1_ref, b1_ref, w2_ref, b2_ref, z_ref, s_ref, ss_ref)


def _pool_outs(r, k2, blk=_BLK):
    return (
        (
            pl.BlockSpec((blk, k2), lambda i: (i, 0)),
            pl.BlockSpec((1, k2), lambda i: (0, 0)),
            pl.BlockSpec((1, k2), lambda i: (0, 0)),
        ),
        (
            jax.ShapeDtypeStruct((r, k2), _F32),
            jax.ShapeDtypeStruct((1, k2), _F32),
            jax.ShapeDtypeStruct((1, k2), _F32),
        ),
    )


def _mlp_specs(mp):
    return [_full(mp['W1'].shape), _full((1, mp['W1'].shape[1])),
            _full(mp['W2'].shape), _full((1, mp['W2'].shape[1]))]


def _mlp_args(mp):
    return (mp['W1'], mp['b1'].reshape(1, -1), mp['W2'],
            mp['b2'].reshape(1, -1))


def _p1(a8, h, mp, blk=_BLK):
    r, cdim = a8.shape
    out_specs, out_shape = _pool_outs(r, mp['W2'].shape[1], blk)
    return pl.pallas_call(
        _p1_body,
        grid=(r // blk,),
        in_specs=[_rowblk(cdim, blk), _full(h.shape)] + _mlp_specs(mp),
        out_specs=out_specs,
        out_shape=out_shape,
        scratch_shapes=[pltpu.VMEM(h.shape, _BF)],
    )(a8, h, *_mlp_args(mp))


def _p2(a8, zp, sp, ssp, bp, mp, blk=_BLK):
    r, cdim = a8.shape
    f = zp.shape[1]
    out_specs, out_shape = _pool_outs(r, mp['W2'].shape[1], blk)
    return pl.pallas_call(
        functools.partial(_p2_body, float(cdim)),
        grid=(r // blk,),
        in_specs=[_rowblk(cdim, blk), _full(zp.shape), _full((1, f)),
                  _full((1, f)), _full((1, f)), _full((1, f))]
        + _mlp_specs(mp),
        out_specs=out_specs,
        out_shape=out_shape,
        scratch_shapes=[pltpu.VMEM((cdim, f), _BF)],
    )(a8, zp, sp, ssp, bp['gamma'].reshape(1, -1),
      bp['beta'].reshape(1, -1), *_mlp_args(mp))


def _fin_body(h, n, blk, z_ref, s_ref, ss_ref, g_ref, bt_ref, wn_ref, an_ref,
              mask_ref, o_ref, wa_s, wm_s, a_s, c_s, b_s, d_s, hbn_s):
    i = pl.program_id(0)

    @pl.when(i == 0)
    def _():
        hbn_s[...] = _bn_relu_expr(z_ref[...], s_ref[...], ss_ref[...],
                                   g_ref[...], bt_ref[...], n)
        _side_prologue(hbn_s[...], wn_ref[...], an_ref[...], h,
                       wa_s, wm_s, a_s, c_s, b_s, d_s)

    out = _att_blk(i, mask_ref[...].astype(_BF), h, blk,
                   wa_s, wm_s, a_s, c_s, b_s, d_s)
    out = _elu(out)
    out = out - jnp.max(out, axis=1, keepdims=True)
    out = out - jnp.log(jnp.sum(jnp.exp(out), axis=1, keepdims=True))
    o_ref[...] = out


def _fin(mask8, z, s, ss, bp, wn, an, blk=_BLK):
    r, cdim = mask8.shape
    f = z.shape[1]
    h = wn.shape[1]
    return pl.pallas_call(
        functools.partial(_fin_body, h, float(r), blk),
        grid=(r // blk,),
        in_specs=[_full(z.shape), _full((1, f)), _full((1, f)),
                  _full((1, f)), _full((1, f)), _full(wn.shape),
                  _full(an.shape), _rowblk(cdim, blk)],
        out_specs=_rowblk(h, blk),
        out_shape=jax.ShapeDtypeStruct((r, h), _F32),
        scratch_shapes=_scratch_side(cdim, h)
        + [pltpu.VMEM((r, f), _F32)],
    )(z, s, ss, bp['gamma'].reshape(1, -1), bp['beta'].reshape(1, -1),
      wn, an, mask8)


def kernel(x, e_x, adj, e_adj, n_e_adj, params):
    nea8 = n_e_adj.astype(jnp.int4)
    p1, p2, p3 = params['in_att'], params['att0'], params['out_att']

    hp1, adj8 = _attp(adj, x, p1['Wn'], p1['an'], cast=True, blk=1024)
    ep1, eo1, eadj8 = _attcatp(e_adj, e_x, p1['We'], p1['ae'], nea8, hp1,
                               cast=True)
    xo1 = _cat(nea8, hp1, ep1, blk=1024)

    hp2, = _attp(adj8, xo1, p2['Wn'], p2['an'], blk=1024)
    ep2, eo2 = _attcatp(eadj8, eo1, p2['We'], p2['ae'], nea8, hp2, blk=1024)
    xo2 = _cat(nea8, hp2, ep2, blk=1024)

    zn, sn, ssn = _p1(adj8, xo2, params['mlp0'], blk=1024)
    ze, se, sse = _p1(eadj8, eo2, params['mlp0'], blk=1024)
    zn, sn2, ssn2 = _p2(adj8, zn, sn, ssn, params['bn0'], params['mlp1'], blk=1024)
    ze, se2, sse2 = _p2(eadj8, ze, se, sse, params['bn0'], params['mlp1'], blk=1024)

    fx = _fin(adj8, zn, sn2, ssn2, params['bn1'], p3['Wn'], p3['an'], blk=1024)
    fe = _fin(eadj8, ze, se2, sse2, params['bn1'], p3['We'], p3['ae'], blk=1024)
    return fx, fe
